# baseline (speedup 1.0000x reference)
import jax
import jax.numpy as jnp
from jax.experimental import pallas as pl

_KB = 2000


def _body(xa_ref, xb_ref, emb_ref, out_ref):
    e = emb_ref[...].astype(jnp.bfloat16)
    dn = (((0,), (0,)), ((), ()))
    pa = jax.lax.dot_general(
        xa_ref[...].astype(jnp.bfloat16), e, dn,
        preferred_element_type=jnp.float32,
    )
    pb = jax.lax.dot_general(
        xb_ref[...].astype(jnp.bfloat16), e, dn,
        preferred_element_type=jnp.float32,
    )
    p = jnp.concatenate([pa, pb], axis=0)

    @pl.when(pl.program_id(0) == 0)
    def _():
        out_ref[...] = p

    @pl.when(pl.program_id(0) != 0)
    def _():
        out_ref[...] += p


def kernel(x_seq, emb):
    B, K = x_seq.shape
    H = emb.shape[1]
    hb = B // 2
    return pl.pallas_call(
        _body,
        grid=(K // _KB,),
        in_specs=[
            pl.BlockSpec((_KB, hb), lambda i: (i, 0)),
            pl.BlockSpec((_KB, hb), lambda i: (i, 1)),
            pl.BlockSpec((_KB, H), lambda i: (i, 0)),
        ],
        out_specs=pl.BlockSpec((B, H), lambda i: (0, 0)),
        out_shape=jax.ShapeDtypeStruct((B, H), jnp.float32),
    )(x_seq.T, x_seq.T, emb)
